# baseline (speedup 1.0000x reference)
import functools

import jax
import jax.numpy as jnp
from jax import lax
from jax.experimental import pallas as pl
from jax.experimental.pallas import tpu as pltpu
from jax.experimental.pallas import tpu_sc as plsc

NC, NS, L = 2, 16, 16
NW = NC * NS
B = 1024
FD, FS = 13, 26
F = FD + FS
K = 16
VOCAB = 1000
FEAT = FD + FS * VOCAB
FEATP = 26112
RPW = B // NW
NM = FS * 5
CH = 6528

_mesh = plsc.VectorSubcoreMesh(
    core_axis_name="c", subcore_axis_name="s", num_cores=NC, num_subcores=NS
)


def _tc_transpose_body(in_ref, out_ref):
    out_ref[...] = in_ref[...].T


@functools.partial(
    pl.kernel,
    out_type=jax.ShapeDtypeStruct((B,), jnp.float32),
    mesh=_mesh,
    compiler_params=pltpu.CompilerParams(
        use_tc_tiling_on_sc=False, needs_layout_passes=False
    ),
    scratch_types=[
        pltpu.VMEM((RPW, F), jnp.float32),
        pltpu.VMEM((FEAT,), jnp.float32),
        pltpu.VMEM((48,), jnp.int32),
        pltpu.VMEM((48,), jnp.int32),
        pltpu.VMEM((65,), jnp.int32),
        pltpu.VMEM((65,), jnp.int32),
        pltpu.VMEM((65,), jnp.int32),
        pltpu.VMEM((2, NM, 128), jnp.float32),
        pltpu.VMEM((65, 128), jnp.float32),
        pltpu.VMEM((16, K), jnp.float32),
        pltpu.VMEM((RPW,), jnp.float32),
        pltpu.VMEM((L,), jnp.float32),
        pltpu.SemaphoreType.DMA,
        pltpu.SemaphoreType.DMA,
    ],
)
def _ffm_sc(x_hbm, w5_hbm, wfc_hbm, bias_hbm, out_hbm,
            xch, wfcv, otab, fcidx, idxa, idxb, udidx, rows2, udv, dbuf,
            zbuf, bv, sem, sem2):
    wid = lax.axis_index("s") * NC + lax.axis_index("c")
    base = wid * RPW
    iota = lax.iota(jnp.int32, L)
    zf = jnp.zeros((L,), jnp.float32)

    pltpu.sync_copy(x_hbm.at[pl.ds(base, RPW)], xch)
    pltpu.sync_copy(bias_hbm, bv)
    pltpu.sync_copy(wfc_hbm, wfcv)

    for k in range(3):
        n = iota + 16 * k
        otab[pl.ds(16 * k, 16)] = jnp.where(
            n < FD, n, jnp.where(n < F, FD + (n - FD) * VOCAB, 0))

    for v in range(5):
        m = iota + 16 * v
        mq = m // 5
        val = (m - 5 * mq) * FEATP + mq
        plsc.store_scatter(udidx, [jnp.minimum(m, 64)], val, mask=m < 65)
    pltpu.async_copy(w5_hbm.at[udidx], udv, sem).wait()

    ic = jnp.minimum(iota, FD - 1)
    icq = ic // 8
    icc = (ic - 8 * icq) * 16

    def _drow(j, _):
        jq = j // 8
        jc = (j - 8 * jq) * 16

        def _dk(k, acc):
            a = plsc.load_gather(udv, [j * 5 + icq, icc + k])
            b = plsc.load_gather(udv, [ic * 5 + jq, jnp.full((L,), jc + k,
                                                             jnp.int32)])
            return acc + a * b
        accd = lax.fori_loop(0, K, _dk, zf)
        dbuf[j] = jnp.where(iota < FD, accd, 0.0)
        return _
    lax.fori_loop(0, FD, _drow, 0)

    def _build_and_fire(rr, dstbuf):
        rfull = jnp.full((L,), rr, jnp.int32)
        for k in range(3):
            fcidx[pl.ds(16 * k, 16)] = otab[pl.ds(16 * k, 16)]
        g1 = plsc.load_gather(xch, [rfull, jnp.minimum(iota + FD, F - 1)])
        g2 = plsc.load_gather(xch, [rfull, jnp.minimum(iota + FD + 16, F - 1)])
        plsc.addupdate_scatter(fcidx, [iota + FD], g1.astype(jnp.int32))
        plsc.addupdate_scatter(fcidx, [iota + FD + 16], g2.astype(jnp.int32),
                               mask=iota < (F - FD - 16))
        for v in range(9):
            m = iota + 16 * v
            mq = jnp.minimum(m // 5, FS)
            feat = plsc.load_gather(fcidx, [FD + mq])
            val = (m - 5 * (m // 5)) * FEATP + feat
            plsc.store_scatter(idxa, [jnp.minimum(m, 64)], val, mask=m < 65)
            plsc.store_scatter(idxb, [jnp.clip(m - 65, 0, 64)], val,
                               mask=jnp.logical_and(m >= 65, m < NM))
        pltpu.async_copy(w5_hbm.at[idxa],
                         rows2.at[dstbuf, pl.ds(0, 65)], sem)
        pltpu.async_copy(w5_hbm.at[idxb],
                         rows2.at[dstbuf, pl.ds(65, 65)], sem2)

    _build_and_fire(0, 0)

    def _row(r, _):
        par = lax.rem(r, 2)
        rfull = jnp.full((L,), r, jnp.int32)

        f0 = plsc.load_gather(wfcv, [plsc.load_gather(fcidx, [iota])])
        f1 = plsc.load_gather(wfcv, [plsc.load_gather(fcidx, [iota + 16])])
        f2 = plsc.load_gather(wfcv, [plsc.load_gather(fcidx, [iota + 32])])
        f2 = jnp.where(iota < F - 32, f2, 0.0)
        linv = f0 + f1 + f2 + jnp.where(iota < 1, bv[...], 0.0)

        pltpu.make_async_copy(w5_hbm.at[idxa],
                              rows2.at[par, pl.ds(0, 65)], sem).wait()
        pltpu.make_async_copy(w5_hbm.at[idxb],
                              rows2.at[par, pl.ds(65, 65)], sem2).wait()

        @pl.when(r < RPW - 1)
        def _fire_next():
            _build_and_fire(r + 1, 1 - par)

        xd = plsc.load_gather(xch, [rfull, ic])
        xd = jnp.where(iota < FD, xd, 0.0)
        accv = linv
        for j in range(FD):
            bx = plsc.load_gather(xch, [rfull, jnp.full((L,), j, jnp.int32)])
            accv = accv + bx * (dbuf[j] * xd)

        for d in range(FD):
            dq, dc = d // 8, (d % 8) * 16

            @plsc.parallel_loop(0, FS - 1, carry=zf, unroll=5)
            def _dsb(sp, a, dq=dq, dc=dc, d=d, par=par):
                i = FD + sp
                iq = i // 8
                icl = (i - 8 * iq) * 16
                t = (rows2[par, sp * 5 + dq, pl.ds(dc, 16)]
                     * udv[d * 5 + iq, pl.ds(icl, 16)])
                return a + (t + t)
            t38 = (rows2[par, (FS - 1) * 5 + dq, pl.ds(dc, 16)]
                   * udv[d * 5 + (F - 1) // 8, pl.ds(((F - 1) % 8) * 16, 16)])
            bx = plsc.load_gather(xch, [rfull, jnp.full((L,), d, jnp.int32)])
            accv = accv + bx * (_dsb + t38)

        acc_ss = zf
        for jp in range(1, FS):
            j = FD + jp
            jq, jc = j // 8, (j % 8) * 16
            dbl = jp < FS - 1

            @plsc.parallel_loop(0, jp, carry=acc_ss,
                                unroll=4 if jp >= 8 else 1)
            def _ssb(ip, acc, jp=jp, jq=jq, jc=jc, dbl=dbl, par=par):
                i = FD + ip
                iq = i // 8
                icl = (i - 8 * iq) * 16
                va = rows2[par, jp * 5 + iq, pl.ds(icl, 16)]
                vb = rows2[par, ip * 5 + jq, pl.ds(jc, 16)]
                t = va * vb
                return acc + (t + t if dbl else t)
            acc_ss = _ssb

        @plsc.parallel_loop(0, FS - 1, carry=zf, unroll=5)
        def _diag(jp, acc, par=par):
            j = FD + jp
            jq = j // 8
            jc = (j - 8 * jq) * 16
            dg = rows2[par, jp * 5 + jq, pl.ds(jc, 16)]
            return acc + dg * dg
        accv = accv + acc_ss + _diag

        z = jnp.sum(accv)
        plsc.store_scatter(zbuf, [rfull], jnp.full((L,), z), mask=iota < 1)
        return _
    lax.fori_loop(0, RPW, _row, 0)

    for k in range(2):
        zv = zbuf[pl.ds(k * 16, 16)]
        zbuf[pl.ds(k * 16, 16)] = 1.0 / (1.0 + jnp.exp(-zv))
    pltpu.sync_copy(zbuf, out_hbm.at[pl.ds(base, RPW)])


def kernel(x, W_emb, W_fc, bias):
    wn2d = jnp.transpose(W_emb, (0, 2, 1)).reshape(F * K, FEAT)
    t5 = pl.pallas_call(
        _tc_transpose_body,
        grid=(5, FEATP // CH),
        in_specs=[pl.BlockSpec((128, CH), lambda q, c: (q, c))],
        out_specs=pl.BlockSpec((CH, 128), lambda q, c: (q * (FEATP // CH) + c,
                                                        0)),
        out_shape=jax.ShapeDtypeStruct((5 * FEATP, 128), jnp.float32),
    )(wn2d)
    wfc_flat = W_fc.reshape(FEAT)
    bias16 = jnp.broadcast_to(bias, (L,))
    out = _ffm_sc(x, t5, wfc_flat, bias16)
    return out.reshape(B, 1)


# speedup vs baseline: 1.0259x; 1.0259x over previous
import functools

import jax
import jax.numpy as jnp
from jax import lax
from jax.experimental import pallas as pl
from jax.experimental.pallas import tpu as pltpu
from jax.experimental.pallas import tpu_sc as plsc

NC, NS, L = 2, 16, 16
NW = NC * NS
B = 1024
FD, FS = 13, 26
F = FD + FS
K = 16
VOCAB = 1000
FEAT = FD + FS * VOCAB
FEATP = 26112
RPW = B // NW
NM = FS * 5
CH = 26112

_mesh = plsc.VectorSubcoreMesh(
    core_axis_name="c", subcore_axis_name="s", num_cores=NC, num_subcores=NS
)


def _tc_transpose_body(in_ref, out_ref):
    out_ref[...] = in_ref[...].T


@functools.partial(
    pl.kernel,
    out_type=jax.ShapeDtypeStruct((B,), jnp.float32),
    mesh=_mesh,
    compiler_params=pltpu.CompilerParams(
        use_tc_tiling_on_sc=False, needs_layout_passes=False
    ),
    scratch_types=[
        pltpu.VMEM((RPW, F), jnp.float32),
        pltpu.VMEM((FEAT,), jnp.float32),
        pltpu.VMEM((48,), jnp.int32),
        pltpu.VMEM((48,), jnp.int32),
        pltpu.VMEM((65,), jnp.int32),
        pltpu.VMEM((65,), jnp.int32),
        pltpu.VMEM((65,), jnp.int32),
        pltpu.VMEM((2, NM, 128), jnp.float32),
        pltpu.VMEM((65, 128), jnp.float32),
        pltpu.VMEM((16, K), jnp.float32),
        pltpu.VMEM((RPW,), jnp.float32),
        pltpu.VMEM((L,), jnp.float32),
        pltpu.SemaphoreType.DMA,
        pltpu.SemaphoreType.DMA,
    ],
)
def _ffm_sc(x_hbm, w5_hbm, wfc_hbm, bias_hbm, out_hbm,
            xch, wfcv, otab, fcidx, idxa, idxb, udidx, rows2, udv, dbuf,
            zbuf, bv, sem, sem2):
    wid = lax.axis_index("s") * NC + lax.axis_index("c")
    base = wid * RPW
    iota = lax.iota(jnp.int32, L)
    zf = jnp.zeros((L,), jnp.float32)

    pltpu.sync_copy(x_hbm.at[pl.ds(base, RPW)], xch)
    pltpu.sync_copy(bias_hbm, bv)
    pltpu.sync_copy(wfc_hbm, wfcv)

    for k in range(3):
        n = iota + 16 * k
        otab[pl.ds(16 * k, 16)] = jnp.where(
            n < FD, n, jnp.where(n < F, FD + (n - FD) * VOCAB, 0))

    for v in range(5):
        m = iota + 16 * v
        mq = m // 5
        val = (m - 5 * mq) * FEATP + mq
        plsc.store_scatter(udidx, [jnp.minimum(m, 64)], val, mask=m < 65)
    pltpu.async_copy(w5_hbm.at[udidx], udv, sem).wait()

    ic = jnp.minimum(iota, FD - 1)
    icq = ic // 8
    icc = (ic - 8 * icq) * 16

    def _drow(j, _):
        jq = j // 8
        jc = (j - 8 * jq) * 16

        def _dk(k, acc):
            a = plsc.load_gather(udv, [j * 5 + icq, icc + k])
            b = plsc.load_gather(udv, [ic * 5 + jq, jnp.full((L,), jc + k,
                                                             jnp.int32)])
            return acc + a * b
        accd = lax.fori_loop(0, K, _dk, zf)
        dbuf[j] = jnp.where(iota < FD, accd, 0.0)
        return _
    lax.fori_loop(0, FD, _drow, 0)

    def _build_and_fire(rr, dstbuf):
        rfull = jnp.full((L,), rr, jnp.int32)
        for k in range(3):
            fcidx[pl.ds(16 * k, 16)] = otab[pl.ds(16 * k, 16)]
        g1 = plsc.load_gather(xch, [rfull, jnp.minimum(iota + FD, F - 1)])
        g2 = plsc.load_gather(xch, [rfull, jnp.minimum(iota + FD + 16, F - 1)])
        plsc.addupdate_scatter(fcidx, [iota + FD], g1.astype(jnp.int32))
        plsc.addupdate_scatter(fcidx, [iota + FD + 16], g2.astype(jnp.int32),
                               mask=iota < (F - FD - 16))
        for v in range(9):
            m = iota + 16 * v
            mq = jnp.minimum(m // 5, FS)
            feat = plsc.load_gather(fcidx, [FD + mq])
            val = (m - 5 * (m // 5)) * FEATP + feat
            plsc.store_scatter(idxa, [jnp.minimum(m, 64)], val, mask=m < 65)
            plsc.store_scatter(idxb, [jnp.clip(m - 65, 0, 64)], val,
                               mask=jnp.logical_and(m >= 65, m < NM))
        pltpu.async_copy(w5_hbm.at[idxa],
                         rows2.at[dstbuf, pl.ds(0, 65)], sem)
        pltpu.async_copy(w5_hbm.at[idxb],
                         rows2.at[dstbuf, pl.ds(65, 65)], sem2)

    _build_and_fire(0, 0)

    def _row(r, _):
        par = lax.rem(r, 2)
        rfull = jnp.full((L,), r, jnp.int32)

        f0 = plsc.load_gather(wfcv, [plsc.load_gather(fcidx, [iota])])
        f1 = plsc.load_gather(wfcv, [plsc.load_gather(fcidx, [iota + 16])])
        f2 = plsc.load_gather(wfcv, [plsc.load_gather(fcidx, [iota + 32])])
        f2 = jnp.where(iota < F - 32, f2, 0.0)
        linv = f0 + f1 + f2 + jnp.where(iota < 1, bv[...], 0.0)

        pltpu.make_async_copy(w5_hbm.at[idxa],
                              rows2.at[par, pl.ds(0, 65)], sem).wait()
        pltpu.make_async_copy(w5_hbm.at[idxb],
                              rows2.at[par, pl.ds(65, 65)], sem2).wait()

        @pl.when(r < RPW - 1)
        def _fire_next():
            _build_and_fire(r + 1, 1 - par)

        xd = plsc.load_gather(xch, [rfull, ic])
        xd = jnp.where(iota < FD, xd, 0.0)
        accv = linv
        for j in range(FD):
            bx = plsc.load_gather(xch, [rfull, jnp.full((L,), j, jnp.int32)])
            accv = accv + bx * (dbuf[j] * xd)

        for d in range(FD):
            dq, dc = d // 8, (d % 8) * 16

            @plsc.parallel_loop(0, FS - 1, carry=zf, unroll=5)
            def _dsb(sp, a, dq=dq, dc=dc, d=d, par=par):
                i = FD + sp
                iq = i // 8
                icl = (i - 8 * iq) * 16
                t = (rows2[par, sp * 5 + dq, pl.ds(dc, 16)]
                     * udv[d * 5 + iq, pl.ds(icl, 16)])
                return a + (t + t)
            t38 = (rows2[par, (FS - 1) * 5 + dq, pl.ds(dc, 16)]
                   * udv[d * 5 + (F - 1) // 8, pl.ds(((F - 1) % 8) * 16, 16)])
            bx = plsc.load_gather(xch, [rfull, jnp.full((L,), d, jnp.int32)])
            accv = accv + bx * (_dsb + t38)

        acc_ss = zf
        for jp in range(1, FS):
            j = FD + jp
            jq, jc = j // 8, (j % 8) * 16
            dbl = jp < FS - 1

            @plsc.parallel_loop(0, jp, carry=acc_ss,
                                unroll=4 if jp >= 8 else 1)
            def _ssb(ip, acc, jp=jp, jq=jq, jc=jc, dbl=dbl, par=par):
                i = FD + ip
                iq = i // 8
                icl = (i - 8 * iq) * 16
                va = rows2[par, jp * 5 + iq, pl.ds(icl, 16)]
                vb = rows2[par, ip * 5 + jq, pl.ds(jc, 16)]
                t = va * vb
                return acc + (t + t if dbl else t)
            acc_ss = _ssb

        @plsc.parallel_loop(0, FS - 1, carry=zf, unroll=5)
        def _diag(jp, acc, par=par):
            j = FD + jp
            jq = j // 8
            jc = (j - 8 * jq) * 16
            dg = rows2[par, jp * 5 + jq, pl.ds(jc, 16)]
            return acc + dg * dg
        accv = accv + acc_ss + _diag

        z = jnp.sum(accv)
        plsc.store_scatter(zbuf, [rfull], jnp.full((L,), z), mask=iota < 1)
        return _
    lax.fori_loop(0, RPW, _row, 0)

    for k in range(2):
        zv = zbuf[pl.ds(k * 16, 16)]
        zbuf[pl.ds(k * 16, 16)] = 1.0 / (1.0 + jnp.exp(-zv))
    pltpu.sync_copy(zbuf, out_hbm.at[pl.ds(base, RPW)])


def kernel(x, W_emb, W_fc, bias):
    wn2d = jnp.transpose(W_emb, (0, 2, 1)).reshape(F * K, FEAT)
    t5 = pl.pallas_call(
        _tc_transpose_body,
        grid=(5, FEATP // CH),
        in_specs=[pl.BlockSpec((128, CH), lambda q, c: (q, c))],
        out_specs=pl.BlockSpec((CH, 128), lambda q, c: (q * (FEATP // CH) + c,
                                                        0)),
        out_shape=jax.ShapeDtypeStruct((5 * FEATP, 128), jnp.float32),
    )(wn2d)
    wfc_flat = W_fc.reshape(FEAT)
    bias16 = jnp.broadcast_to(bias, (L,))
    out = _ffm_sc(x, t5, wfc_flat, bias16)
    return out.reshape(B, 1)
